# baseline (speedup 1.0000x reference)
import functools

import numpy as np
import jax
import jax.numpy as jnp
from jax import lax
from jax.experimental import pallas as pl
from jax.experimental.pallas import tpu as pltpu
from jax.experimental.pallas import tpu_sc as plsc

DIM = 256
K = 8192
N = 9216

BN = 4608
NB = N // BN

KC = 256
NKC = K // KC


def _argmin_body(xf_ref, e_ref, xsq_ref, esq_ref, out_ref):
    x2 = xf_ref[...] * 2.0
    xsq = xsq_ref[...]
    e = e_ref[...]
    esq = esq_ref[...]
    run = None
    cid = None
    for c in range(NKC):
        e_c = lax.slice(e, (c * KC, 0), ((c + 1) * KC, DIM))
        dot2 = lax.dot_general(
            x2, e_c,
            dimension_numbers=(((1,), (1,)), ((), ())),
            preferred_element_type=jnp.float32,
        )
        t_c = xsq - dot2 + lax.slice(esq, (0, c * KC), (1, (c + 1) * KC))
        if run is None:
            run = t_c
            cid = jnp.zeros(t_c.shape, jnp.int32)
        else:
            lt = t_c < run
            run = jnp.minimum(t_c, run)
            cid = jnp.where(lt, jnp.int32(c), cid)
    m = jnp.min(run, axis=1, keepdims=True)
    lane = lax.broadcasted_iota(jnp.int32, run.shape, 1)
    idxc = jnp.where(run == m, cid * KC + lane, jnp.int32(K))
    out_ref[0, 0, :] = jnp.min(idxc, axis=1)


def _nearest_indices(xf, eT, xsq, esq):
    out = pl.pallas_call(
        _argmin_body,
        grid=(NB,),
        in_specs=[
            pl.BlockSpec((BN, DIM), lambda i: (i, 0)),
            pl.BlockSpec((K, DIM), lambda i: (0, 0)),
            pl.BlockSpec((BN, 1), lambda i: (i, 0)),
            pl.BlockSpec((1, K), lambda i: (0, 0)),
        ],
        out_specs=pl.BlockSpec((1, 1, BN), lambda i: (i, 0, 0)),
        out_shape=jax.ShapeDtypeStruct((NB, 1, BN), jnp.int32),
    )(xf, eT, xsq, esq)
    return out.reshape(N)


def _gather_rows(table, ind):
    info = plsc.get_sparse_core_info()
    nc, ns = info.num_cores, info.num_subcores
    nw = nc * ns
    b_per_w = N // nw
    mesh = plsc.VectorSubcoreMesh(core_axis_name="c", subcore_axis_name="s")

    @functools.partial(
        pl.kernel,
        mesh=mesh,
        out_type=jax.ShapeDtypeStruct((N, DIM), jnp.float32),
        scratch_types=[
            pltpu.VMEM((b_per_w,), jnp.int32),
            pltpu.VMEM((b_per_w, DIM), jnp.float32),
            pltpu.SemaphoreType.DMA,
        ],
    )
    def gather_k(table_hbm, idx_hbm, out_hbm, idx_v, rows_v, sem):
        wid = lax.axis_index("s") * nc + lax.axis_index("c")
        base = wid * b_per_w
        pltpu.sync_copy(idx_hbm.at[pl.ds(base, b_per_w)], idx_v)
        pltpu.async_copy(table_hbm.at[idx_v], rows_v, sem).wait()
        pltpu.sync_copy(rows_v, out_hbm.at[pl.ds(base, b_per_w)])

    return gather_k(table, ind)


def kernel(x, embed):
    xf = x[0]
    e0 = embed[0]
    xsq = jnp.sum(xf * xf, axis=-1, keepdims=True)
    esq = jnp.sum(e0 * e0, axis=-1)[None, :]
    ind = _nearest_indices(xf, e0, xsq, esq)
    quantize = _gather_rows(e0, ind)
    return quantize, ind.reshape(1, N)


# speedup vs baseline: 1.1776x; 1.1776x over previous
import functools

import numpy as np
import jax
import jax.numpy as jnp
from jax import lax
from jax.experimental import pallas as pl
from jax.experimental.pallas import tpu as pltpu
from jax.experimental.pallas import tpu_sc as plsc

DIM = 256
K = 8192
N = 9216

BN = 3072
NB = N // BN

KC = 256
NKC = K // KC


def _argmin_body(xf_ref, e_ref, xsq_ref, esq_ref, out_ref):
    x2 = xf_ref[...] * 2.0
    xsq = xsq_ref[...]
    e = e_ref[...]
    esq = esq_ref[...]
    run = None
    cid = None
    for c in range(NKC):
        e_c = lax.slice(e, (c * KC, 0), ((c + 1) * KC, DIM))
        dot2 = lax.dot_general(
            x2, e_c,
            dimension_numbers=(((1,), (1,)), ((), ())),
            preferred_element_type=jnp.float32,
        )
        t_c = xsq - dot2 + lax.slice(esq, (0, c * KC), (1, (c + 1) * KC))
        if run is None:
            run = t_c
            cid = jnp.zeros(t_c.shape, jnp.int32)
        else:
            lt = t_c < run
            run = jnp.minimum(t_c, run)
            cid = jnp.where(lt, jnp.int32(c), cid)
    m = jnp.min(run, axis=1, keepdims=True)
    lane = lax.broadcasted_iota(jnp.int32, run.shape, 1)
    idxc = jnp.where(run == m, cid * KC + lane, jnp.int32(K))
    out_ref[0, 0, :] = jnp.min(idxc, axis=1)


def _nearest_indices(xf, eT, xsq, esq):
    out = pl.pallas_call(
        _argmin_body,
        grid=(NB,),
        in_specs=[
            pl.BlockSpec((BN, DIM), lambda i: (i, 0)),
            pl.BlockSpec((K, DIM), lambda i: (0, 0)),
            pl.BlockSpec((BN, 1), lambda i: (i, 0)),
            pl.BlockSpec((1, K), lambda i: (0, 0)),
        ],
        out_specs=pl.BlockSpec((1, 1, BN), lambda i: (i, 0, 0)),
        out_shape=jax.ShapeDtypeStruct((NB, 1, BN), jnp.int32),
    )(xf, eT, xsq, esq)
    return out.reshape(N)


def _gather_rows(table, ind):
    info = plsc.get_sparse_core_info()
    nc, ns = info.num_cores, info.num_subcores
    nw = nc * ns
    b_per_w = N // nw
    mesh = plsc.VectorSubcoreMesh(core_axis_name="c", subcore_axis_name="s")

    @functools.partial(
        pl.kernel,
        mesh=mesh,
        out_type=jax.ShapeDtypeStruct((N, DIM), jnp.float32),
        scratch_types=[
            pltpu.VMEM((b_per_w,), jnp.int32),
            pltpu.VMEM((b_per_w, DIM), jnp.float32),
            pltpu.SemaphoreType.DMA,
        ],
    )
    def gather_k(table_hbm, idx_hbm, out_hbm, idx_v, rows_v, sem):
        wid = lax.axis_index("s") * nc + lax.axis_index("c")
        base = wid * b_per_w
        pltpu.sync_copy(idx_hbm.at[pl.ds(base, b_per_w)], idx_v)
        pltpu.async_copy(table_hbm.at[idx_v], rows_v, sem).wait()
        pltpu.sync_copy(rows_v, out_hbm.at[pl.ds(base, b_per_w)])

    return gather_k(table, ind)


def kernel(x, embed):
    xf = x[0]
    e0 = embed[0]
    xsq = jnp.sum(xf * xf, axis=-1, keepdims=True)
    esq = jnp.sum(e0 * e0, axis=-1)[None, :]
    ind = _nearest_indices(xf, e0, xsq, esq)
    quantize = _gather_rows(e0, ind)
    return quantize, ind.reshape(1, N)
